# Initial kernel scaffold; baseline (speedup 1.0000x reference)
import jax
import jax.numpy as jnp
from jax.experimental import pallas as pl


def kernel(codec, mask_pos, pos_emb, type_emb, gW1, gb1, gW2, gb2, EW1, Eb1, EW2, Eb2, EW3, Eb3, NW, Nb, CW, Cb):
    raise NotImplementedError("write your pallas kernel here")


# speedup vs baseline: 2.4303x; 2.4303x over previous
import jax
import jax.numpy as jnp
from jax import lax
from jax.experimental import pallas as pl
from jax.experimental.pallas import tpu as pltpu

_B = 1024
_C = 256
_S = 39
_NN = 13
_NC = 26
_D = 128
_E = 16
_K = 2
_H = 512
_V = 1000
_G1 = 256

_SQRT_HALF = 0.7071067811865476


def _gelu(x):
    return 0.5 * x * (1.0 + lax.erf(x * _SQRT_HALF))


def _leaky(x):
    return jnp.where(x >= 0, x, 0.01 * x)


def _gate_body(codec_ref, maskf_ref, pe_ref, te_ref, gW1_ref, gb1_ref,
               gW2_ref, gb2_ref, probs_ref, experts_ref, wcomb_ref):
    x = codec_ref[...]
    bt = x.shape[0]
    W1 = gW1_ref[...]
    pe = pe_ref[...]
    te = te_ref[...]
    s_iota = lax.broadcasted_iota(jnp.int32, (_S, _C), 0)
    te_sel = jnp.where(s_iota < _NN, te[0:1, :], te[1:2, :])
    m = maskf_ref[...]
    gi = jnp.concatenate([
        jnp.broadcast_to(x[:, None, :], (bt, _S, _C)),
        jnp.broadcast_to(pe[None], (bt, _S, _C)),
        jnp.broadcast_to(te_sel[None], (bt, _S, _C)),
        m[:, :, None],
    ], axis=-1)
    hpre = (lax.dot_general(gi, W1, (((2,), (0,)), ((), ())))
            + gb1_ref[...][None, None, :])
    h = 0.5 * hpre * (1.0 - lax.erf(-hpre * _SQRT_HALF))
    logits = (lax.dot_general(h, gW2_ref[...], (((2,), (0,)), ((), ())))
              + gb2_ref[...][None, None, :])
    logits = logits * m[:, :, None]

    u = lax.bitcast_convert_type(logits, jnp.int32)
    keys = u ^ ((u >> 31) & jnp.int32(0x7FFFFFFF))
    e_iota = lax.broadcasted_iota(jnp.int32, logits.shape, 2)
    imin = jnp.int32(-2147483648)
    m1k = jnp.max(keys, axis=-1, keepdims=True)
    idx1 = jnp.min(jnp.where(keys == m1k, e_iota, _E), axis=-1, keepdims=True)
    k2 = jnp.where(e_iota == idx1, imin, keys)
    m2k = jnp.max(k2, axis=-1, keepdims=True)
    idx2 = jnp.min(jnp.where(k2 == m2k, e_iota, _E), axis=-1, keepdims=True)
    m1 = jnp.max(logits, axis=-1, keepdims=True)
    m2 = jnp.max(jnp.where(e_iota == idx2, logits, -1e30), axis=-1,
                 keepdims=True)
    t = jnp.exp(m2 - m1)
    p1 = 1.0 / (1.0 + t)
    p2 = t / (1.0 + t)
    probs_ref[...] = jnp.concatenate([p1, p2], axis=-1)
    experts_ref[...] = jnp.concatenate([idx1, idx2], axis=-1)
    wcomb_ref[...] = (p1 * (e_iota == idx1).astype(jnp.float32)
                      + p2 * (e_iota == idx2).astype(jnp.float32))


def _run_gate(codec, maskf, pos_emb, type_emb, gW1, gb1, gW2, gb2, bt):
    grid = (_B // bt,)
    full = lambda *shape: pl.BlockSpec(shape, lambda b: (0,) * len(shape))
    return pl.pallas_call(
        _gate_body,
        grid=grid,
        in_specs=[
            pl.BlockSpec((bt, _C), lambda b: (b, 0)),
            pl.BlockSpec((bt, _S), lambda b: (b, 0)),
            full(_S, _C),
            full(2, _C),
            full(3 * _C + 1, _G1),
            full(_G1),
            full(_G1, _E),
            full(_E),
        ],
        out_specs=[
            pl.BlockSpec((bt, _S, _K), lambda b: (b, 0, 0)),
            pl.BlockSpec((bt, _S, _K), lambda b: (b, 0, 0)),
            pl.BlockSpec((bt, _S, _E), lambda b: (b, 0, 0)),
        ],
        out_shape=[
            jax.ShapeDtypeStruct((_B, _S, _K), jnp.float32),
            jax.ShapeDtypeStruct((_B, _S, _K), jnp.int32),
            jax.ShapeDtypeStruct((_B, _S, _E), jnp.float32),
        ],
    )(codec, maskf, pos_emb, type_emb, gW1, gb1, gW2, gb2)


def _expert_body(codec_ref, W1_ref, b1_ref, W2_ref, b2_ref, W3_ref, b3_ref,
                 out_ref):
    x = codec_ref[...]
    h1 = _leaky(jnp.dot(x, W1_ref[0], preferred_element_type=jnp.float32)
                + b1_ref[0])
    h2 = _leaky(jnp.dot(h1, W2_ref[0], preferred_element_type=jnp.float32)
                + b2_ref[0])
    out_ref[0] = (jnp.dot(h2, W3_ref[0], preferred_element_type=jnp.float32)
                  + b3_ref[0])


def _run_experts(codec, EW1, Eb1, EW2, Eb2, EW3, Eb3):
    return pl.pallas_call(
        _expert_body,
        grid=(_E,),
        in_specs=[
            pl.BlockSpec((_B, _C), lambda e: (0, 0)),
            pl.BlockSpec((1, _C, _H), lambda e: (e, 0, 0)),
            pl.BlockSpec((1, 1, _H), lambda e: (e, 0, 0)),
            pl.BlockSpec((1, _H, _H), lambda e: (e, 0, 0)),
            pl.BlockSpec((1, 1, _H), lambda e: (e, 0, 0)),
            pl.BlockSpec((1, _H, _D), lambda e: (e, 0, 0)),
            pl.BlockSpec((1, 1, _D), lambda e: (e, 0, 0)),
        ],
        out_specs=pl.BlockSpec((1, _B, _D), lambda e: (e, 0, 0)),
        out_shape=jax.ShapeDtypeStruct((_E, _B, _D), jnp.float32),
    )(codec, EW1, Eb1[:, None, :], EW2, Eb2[:, None, :], EW3, Eb3[:, None, :])


def _heads_body(wcomb_ref, eo_ref, CW_ref, Cb_ref, NW_ref, Nb_ref,
                cat_ref, num_ref):
    w = wcomb_ref[...]
    bt = w.shape[0]
    combined = jnp.zeros((bt, _S, _D), jnp.float32)
    for e in range(_E):
        combined = combined + w[:, :, e][:, :, None] * eo_ref[e][:, None, :]
    nw = NW_ref[...][:, :, 0]
    num_ref[...] = (jnp.sum(combined[:, :_NN, :] * nw[None], axis=-1)
                    + Nb_ref[...][:, 0][None, :])
    for n in range(_NC):
        cat_ref[:, n, :] = (jnp.dot(combined[:, _NN + n, :], CW_ref[n],
                                    preferred_element_type=jnp.float32)
                            + Cb_ref[n])


def _run_heads(wcomb, eo, CW, Cb, NW, Nb, bt):
    grid = (_B // bt,)
    full = lambda *shape: pl.BlockSpec(shape, lambda b: (0,) * len(shape))
    return pl.pallas_call(
        _heads_body,
        grid=grid,
        in_specs=[
            pl.BlockSpec((bt, _S, _E), lambda b: (b, 0, 0)),
            pl.BlockSpec((_E, bt, _D), lambda b: (0, b, 0)),
            full(_NC, _D, _V),
            full(_NC, 1, _V),
            full(_NN, _D, 1),
            full(_NN, 1),
        ],
        out_specs=[
            pl.BlockSpec((bt, _NC, _V), lambda b: (b, 0, 0)),
            pl.BlockSpec((bt, _NN), lambda b: (b, 0)),
        ],
        out_shape=[
            jax.ShapeDtypeStruct((_B, _NC, _V), jnp.float32),
            jax.ShapeDtypeStruct((_B, _NN), jnp.float32),
        ],
    )(wcomb, eo, CW, Cb[:, None, :], NW, Nb)


def kernel(codec, mask_pos, pos_emb, type_emb, gW1, gb1, gW2, gb2, EW1, Eb1,
           EW2, Eb2, EW3, Eb3, NW, Nb, CW, Cb):
    maskf = mask_pos.astype(jnp.float32)
    gate_probs, topk_experts, wcomb = _run_gate(
        codec, maskf, pos_emb, type_emb, gW1, gb1, gW2, gb2, bt=64)
    eo = _run_experts(codec, EW1, Eb1, EW2, Eb2, EW3, Eb3)
    cat_recon, num_recon = _run_heads(wcomb, eo, CW, Cb, NW, Nb, bt=64)
    feature_indices = jnp.broadcast_to(
        jnp.arange(_S, dtype=jnp.int32)[None, :], (_B, _S))
    return (num_recon, cat_recon, gate_probs, topk_experts, mask_pos,
            feature_indices)


# speedup vs baseline: 2.7778x; 1.1430x over previous
import jax
import jax.numpy as jnp
from jax import lax
from jax.experimental import pallas as pl
from jax.experimental.pallas import tpu as pltpu

_B = 1024
_C = 256
_S = 39
_NN = 13
_NC = 26
_D = 128
_E = 16
_K = 2
_H = 512
_V = 1000
_G1 = 256

_SQRT_HALF = 0.7071067811865476


def _gelu(x):
    return 0.5 * x * (1.0 + lax.erf(x * _SQRT_HALF))


def _leaky(x):
    return jnp.where(x >= 0, x, 0.01 * x)


def _gate_body(codec_ref, maskf_ref, pe_ref, te_ref, gW1_ref, gb1_ref,
               gW2_ref, gb2_ref, probs_ref, experts_ref, wcomb_ref):
    x = codec_ref[...]
    W1 = gW1_ref[...]
    pe = pe_ref[...]
    te = te_ref[...]
    s_iota = lax.broadcasted_iota(jnp.int32, (_S, _C), 0)
    te_sel = jnp.where(s_iota < _NN, te[0:1, :], te[1:2, :])
    m = maskf_ref[...]
    a = jnp.dot(x, W1[0:_C])
    p = (jnp.dot(pe, W1[_C:2 * _C])
         + jnp.dot(te_sel, W1[2 * _C:3 * _C]))
    wm = W1[3 * _C].astype(jnp.bfloat16).astype(jnp.float32)
    hpre = (a[:, None, :] + p[None, :, :]
            + m[:, :, None] * wm[None, None, :]
            + gb1_ref[...][None, None, :])
    h = 0.5 * hpre * (1.0 - lax.erf(-hpre * _SQRT_HALF))
    logits = (lax.dot_general(h, gW2_ref[...], (((2,), (0,)), ((), ())))
              + gb2_ref[...][None, None, :])
    logits = logits * m[:, :, None]

    u = lax.bitcast_convert_type(logits, jnp.int32)
    keys = u ^ ((u >> 31) & jnp.int32(0x7FFFFFFF))
    e_iota = lax.broadcasted_iota(jnp.int32, logits.shape, 2)
    imin = jnp.int32(-2147483648)
    m1k = jnp.max(keys, axis=-1, keepdims=True)
    idx1 = jnp.min(jnp.where(keys == m1k, e_iota, _E), axis=-1, keepdims=True)
    k2 = jnp.where(e_iota == idx1, imin, keys)
    m2k = jnp.max(k2, axis=-1, keepdims=True)
    idx2 = jnp.min(jnp.where(k2 == m2k, e_iota, _E), axis=-1, keepdims=True)
    m1 = jnp.max(logits, axis=-1, keepdims=True)
    m2 = jnp.max(jnp.where(e_iota == idx2, logits, -1e30), axis=-1,
                 keepdims=True)
    t = jnp.exp(m2 - m1)
    p1 = 1.0 / (1.0 + t)
    p2 = t / (1.0 + t)
    probs_ref[...] = jnp.concatenate([p1, p2], axis=-1)
    experts_ref[...] = jnp.concatenate([idx1, idx2], axis=-1)
    wcomb_ref[...] = (p1 * (e_iota == idx1).astype(jnp.float32)
                      + p2 * (e_iota == idx2).astype(jnp.float32))


def _run_gate(codec, maskf, pos_emb, type_emb, gW1, gb1, gW2, gb2, bt):
    grid = (_B // bt,)
    full = lambda *shape: pl.BlockSpec(shape, lambda b: (0,) * len(shape))
    return pl.pallas_call(
        _gate_body,
        grid=grid,
        in_specs=[
            pl.BlockSpec((bt, _C), lambda b: (b, 0)),
            pl.BlockSpec((bt, _S), lambda b: (b, 0)),
            full(_S, _C),
            full(2, _C),
            full(3 * _C + 1, _G1),
            full(_G1),
            full(_G1, _E),
            full(_E),
        ],
        out_specs=[
            pl.BlockSpec((bt, _S, _K), lambda b: (b, 0, 0)),
            pl.BlockSpec((bt, _S, _K), lambda b: (b, 0, 0)),
            pl.BlockSpec((bt, _S, _E), lambda b: (b, 0, 0)),
        ],
        out_shape=[
            jax.ShapeDtypeStruct((_B, _S, _K), jnp.float32),
            jax.ShapeDtypeStruct((_B, _S, _K), jnp.int32),
            jax.ShapeDtypeStruct((_B, _S, _E), jnp.float32),
        ],
    )(codec, maskf, pos_emb, type_emb, gW1, gb1, gW2, gb2)


def _expert_body(codec_ref, W1_ref, b1_ref, W2_ref, b2_ref, W3_ref, b3_ref,
                 out_ref):
    x = codec_ref[...]
    h1 = _leaky(jnp.dot(x, W1_ref[0], preferred_element_type=jnp.float32)
                + b1_ref[0])
    h2 = _leaky(jnp.dot(h1, W2_ref[0], preferred_element_type=jnp.float32)
                + b2_ref[0])
    out_ref[0] = (jnp.dot(h2, W3_ref[0], preferred_element_type=jnp.float32)
                  + b3_ref[0])


def _run_experts(codec, EW1, Eb1, EW2, Eb2, EW3, Eb3):
    return pl.pallas_call(
        _expert_body,
        grid=(_E,),
        in_specs=[
            pl.BlockSpec((_B, _C), lambda e: (0, 0)),
            pl.BlockSpec((1, _C, _H), lambda e: (e, 0, 0)),
            pl.BlockSpec((1, 1, _H), lambda e: (e, 0, 0)),
            pl.BlockSpec((1, _H, _H), lambda e: (e, 0, 0)),
            pl.BlockSpec((1, 1, _H), lambda e: (e, 0, 0)),
            pl.BlockSpec((1, _H, _D), lambda e: (e, 0, 0)),
            pl.BlockSpec((1, 1, _D), lambda e: (e, 0, 0)),
        ],
        out_specs=pl.BlockSpec((1, _B, _D), lambda e: (e, 0, 0)),
        out_shape=jax.ShapeDtypeStruct((_E, _B, _D), jnp.float32),
    )(codec, EW1, Eb1[:, None, :], EW2, Eb2[:, None, :], EW3, Eb3[:, None, :])


def _heads_body(wcomb_ref, eo_ref, CW_ref, Cb_ref, NW_ref, Nb_ref,
                cat_ref, num_ref):
    w = wcomb_ref[...]
    bt = w.shape[0]
    combined = jnp.zeros((bt, _S, _D), jnp.float32)
    for e in range(_E):
        combined = combined + w[:, :, e][:, :, None] * eo_ref[e][:, None, :]
    nw = NW_ref[...][:, :, 0]
    num_ref[...] = (jnp.sum(combined[:, :_NN, :] * nw[None], axis=-1)
                    + Nb_ref[...][:, 0][None, :])
    for n in range(_NC):
        cat_ref[:, n, :] = (jnp.dot(combined[:, _NN + n, :], CW_ref[n],
                                    preferred_element_type=jnp.float32)
                            + Cb_ref[n])


def _run_heads(wcomb, eo, CW, Cb, NW, Nb, bt):
    grid = (_B // bt,)
    full = lambda *shape: pl.BlockSpec(shape, lambda b: (0,) * len(shape))
    return pl.pallas_call(
        _heads_body,
        grid=grid,
        in_specs=[
            pl.BlockSpec((bt, _S, _E), lambda b: (b, 0, 0)),
            pl.BlockSpec((_E, bt, _D), lambda b: (0, b, 0)),
            full(_NC, _D, _V),
            full(_NC, 1, _V),
            full(_NN, _D, 1),
            full(_NN, 1),
        ],
        out_specs=[
            pl.BlockSpec((bt, _NC, _V), lambda b: (b, 0, 0)),
            pl.BlockSpec((bt, _NN), lambda b: (b, 0)),
        ],
        out_shape=[
            jax.ShapeDtypeStruct((_B, _NC, _V), jnp.float32),
            jax.ShapeDtypeStruct((_B, _NN), jnp.float32),
        ],
    )(wcomb, eo, CW, Cb[:, None, :], NW, Nb)


def kernel(codec, mask_pos, pos_emb, type_emb, gW1, gb1, gW2, gb2, EW1, Eb1,
           EW2, Eb2, EW3, Eb3, NW, Nb, CW, Cb):
    maskf = mask_pos.astype(jnp.float32)
    gate_probs, topk_experts, wcomb = _run_gate(
        codec, maskf, pos_emb, type_emb, gW1, gb1, gW2, gb2, bt=128)
    eo = _run_experts(codec, EW1, Eb1, EW2, Eb2, EW3, Eb3)
    cat_recon, num_recon = _run_heads(wcomb, eo, CW, Cb, NW, Nb, bt=64)
    feature_indices = jnp.broadcast_to(
        jnp.arange(_S, dtype=jnp.int32)[None, :], (_B, _S))
    return (num_recon, cat_recon, gate_probs, topk_experts, mask_pos,
            feature_indices)


# speedup vs baseline: 3.2305x; 1.1630x over previous
import functools

import jax
import jax.numpy as jnp
from jax import lax
from jax.experimental import pallas as pl
from jax.experimental.pallas import tpu as pltpu
from jax.experimental.pallas import tpu_sc as plsc

_B = 1024
_C = 256
_S = 39
_NN = 13
_NC = 26
_D = 128
_E = 16
_K = 2
_H = 512
_V = 1000
_G1 = 256

_SQRT_HALF = 0.7071067811865476


def _gelu(x):
    return 0.5 * x * (1.0 + lax.erf(x * _SQRT_HALF))


def _leaky(x):
    return jnp.where(x >= 0, x, 0.01 * x)


def _gate_body(codec_ref, maskf_ref, pe_ref, te_ref, gW1_ref, gb1_ref,
               gW2_ref, gb2_ref, logits_ref):
    x = codec_ref[...]
    W1 = gW1_ref[...]
    pe = pe_ref[...]
    te = te_ref[...]
    s_iota = lax.broadcasted_iota(jnp.int32, (_S, _C), 0)
    te_sel = jnp.where(s_iota < _NN, te[0:1, :], te[1:2, :])
    m = maskf_ref[...]
    a = jnp.dot(x, W1[0:_C])
    p = (jnp.dot(pe, W1[_C:2 * _C])
         + jnp.dot(te_sel, W1[2 * _C:3 * _C]))
    wm = W1[3 * _C].astype(jnp.bfloat16).astype(jnp.float32)
    hpre = (a[:, None, :] + p[None, :, :]
            + m[:, :, None] * wm[None, None, :]
            + gb1_ref[...][None, None, :])
    h = 0.5 * hpre * (1.0 - lax.erf(-hpre * _SQRT_HALF))
    logits = (lax.dot_general(h, gW2_ref[...], (((2,), (0,)), ((), ())))
              + gb2_ref[...][None, None, :])
    logits_ref[...] = logits * m[:, :, None]


def _run_gate(codec, maskf, pos_emb, type_emb, gW1, gb1, gW2, gb2, bt):
    grid = (_B // bt,)
    full = lambda *shape: pl.BlockSpec(shape, lambda b: (0,) * len(shape))
    return pl.pallas_call(
        _gate_body,
        grid=grid,
        in_specs=[
            pl.BlockSpec((bt, _C), lambda b: (b, 0)),
            pl.BlockSpec((bt, _S), lambda b: (b, 0)),
            full(_S, _C),
            full(2, _C),
            full(3 * _C + 1, _G1),
            full(_G1),
            full(_G1, _E),
            full(_E),
        ],
        out_specs=pl.BlockSpec((bt, _S, _E), lambda b: (b, 0, 0)),
        out_shape=jax.ShapeDtypeStruct((_B, _S, _E), jnp.float32),
    )(codec, maskf, pos_emb, type_emb, gW1, gb1, gW2, gb2)


_R = _B * _S
_NW_SC = 32
_RPW = _R // _NW_SC


def _routing_body(logits_hbm, probs_hbm, experts_hbm, wcomb_hbm,
                  lv, pv, ev, wv):
    nc = 2
    wid = lax.axis_index("s") * nc + lax.axis_index("c")
    base = wid * _RPW
    pltpu.sync_copy(logits_hbm.at[pl.ds(base * _E, _RPW * _E)], lv)

    iota16 = lax.iota(jnp.int32, 16)
    imin = jnp.int32(-2147483648)

    def grp(g, carry):
        rowbase = iota16 * _E + g * (16 * _E)
        vs = [plsc.load_gather(lv, [rowbase + e]) for e in range(_E)]
        ks = []
        for e in range(_E):
            u = plsc.bitcast(vs[e], jnp.int32)
            ks.append(u ^ jnp.where(u < 0, jnp.int32(0x7FFFFFFF),
                                    jnp.int32(0)))
        m1 = ks[0]
        for e in range(1, _E):
            m1 = jnp.maximum(m1, ks[e])
        idx1 = jnp.full((16,), _E, jnp.int32)
        for e in range(_E - 1, -1, -1):
            idx1 = jnp.where(ks[e] == m1, jnp.int32(e), idx1)
        k2s = [jnp.where(idx1 == e, imin, ks[e]) for e in range(_E)]
        m2 = k2s[0]
        for e in range(1, _E):
            m2 = jnp.maximum(m2, k2s[e])
        idx2 = jnp.full((16,), _E, jnp.int32)
        for e in range(_E - 1, -1, -1):
            idx2 = jnp.where(k2s[e] == m2, jnp.int32(e), idx2)
        v1 = vs[0]
        for e in range(1, _E):
            v1 = jnp.maximum(v1, vs[e])
        v2 = jnp.full((16,), -1e30, jnp.float32)
        for e in range(_E):
            v2 = jnp.where(idx2 == e, vs[e], v2)
        t = jnp.exp(v2 - v1)
        p1 = 1.0 / (1.0 + t)
        p2 = t / (1.0 + t)
        for e in range(_E):
            we = (jnp.where(idx1 == e, p1, 0.0)
                  + jnp.where(idx2 == e, p2, 0.0))
            plsc.store_scatter(wv, [rowbase + e], we)
        pairbase = iota16 * _K + g * (16 * _K)
        plsc.store_scatter(pv, [pairbase], p1)
        plsc.store_scatter(pv, [pairbase + 1], p2)
        plsc.store_scatter(ev, [pairbase], idx1)
        plsc.store_scatter(ev, [pairbase + 1], idx2)
        return carry

    lax.fori_loop(0, _RPW // 16, grp, 0)
    pltpu.sync_copy(wv, wcomb_hbm.at[pl.ds(base * _E, _RPW * _E)])
    pltpu.sync_copy(pv, probs_hbm.at[pl.ds(base * _K, _RPW * _K)])
    pltpu.sync_copy(ev, experts_hbm.at[pl.ds(base * _K, _RPW * _K)])


def _run_routing(logits):
    mesh = plsc.VectorSubcoreMesh(core_axis_name="c", subcore_axis_name="s")
    f = pl.kernel(
        _routing_body, mesh=mesh,
        compiler_params=pltpu.CompilerParams(needs_layout_passes=False),
        out_type=[
            jax.ShapeDtypeStruct((_R * _K,), jnp.float32),
            jax.ShapeDtypeStruct((_R * _K,), jnp.int32),
            jax.ShapeDtypeStruct((_R * _E,), jnp.float32),
        ],
        scratch_types=[
            pltpu.VMEM((_RPW * _E,), jnp.float32),
            pltpu.VMEM((_RPW * _K,), jnp.float32),
            pltpu.VMEM((_RPW * _K,), jnp.int32),
            pltpu.VMEM((_RPW * _E,), jnp.float32),
        ],
    )
    probs, experts, wcomb = f(logits.reshape(_R * _E))
    return (probs.reshape(_B, _S, _K), experts.reshape(_B, _S, _K),
            wcomb.reshape(_B, _S, _E))


def _expert_body(codec_ref, W1_ref, b1_ref, W2_ref, b2_ref, W3_ref, b3_ref,
                 out_ref):
    x = codec_ref[...]
    h1 = _leaky(jnp.dot(x, W1_ref[0], preferred_element_type=jnp.float32)
                + b1_ref[0])
    h2 = _leaky(jnp.dot(h1, W2_ref[0], preferred_element_type=jnp.float32)
                + b2_ref[0])
    out_ref[0] = (jnp.dot(h2, W3_ref[0], preferred_element_type=jnp.float32)
                  + b3_ref[0])


def _run_experts(codec, EW1, Eb1, EW2, Eb2, EW3, Eb3):
    return pl.pallas_call(
        _expert_body,
        grid=(_E,),
        in_specs=[
            pl.BlockSpec((_B, _C), lambda e: (0, 0)),
            pl.BlockSpec((1, _C, _H), lambda e: (e, 0, 0)),
            pl.BlockSpec((1, 1, _H), lambda e: (e, 0, 0)),
            pl.BlockSpec((1, _H, _H), lambda e: (e, 0, 0)),
            pl.BlockSpec((1, 1, _H), lambda e: (e, 0, 0)),
            pl.BlockSpec((1, _H, _D), lambda e: (e, 0, 0)),
            pl.BlockSpec((1, 1, _D), lambda e: (e, 0, 0)),
        ],
        out_specs=pl.BlockSpec((1, _B, _D), lambda e: (e, 0, 0)),
        out_shape=jax.ShapeDtypeStruct((_E, _B, _D), jnp.float32),
    )(codec, EW1, Eb1[:, None, :], EW2, Eb2[:, None, :], EW3, Eb3[:, None, :])


def _heads_body(wcomb_ref, eo_ref, CW_ref, Cb_ref, NW_ref, Nb_ref,
                cat_ref, num_ref):
    w = wcomb_ref[...]
    eo = eo_ref[...]
    combined = lax.dot_general(w, eo, (((2,), (0,)), ((0,), (1,))),
                               preferred_element_type=jnp.float32)
    nw = NW_ref[...][:, :, 0]
    num_ref[...] = (jnp.sum(combined[:, :_NN, :] * nw[None], axis=-1)
                    + Nb_ref[...][:, 0][None, :])
    for n in range(_NC):
        cat_ref[:, n, :] = (jnp.dot(combined[:, _NN + n, :], CW_ref[n],
                                    preferred_element_type=jnp.float32)
                            + Cb_ref[n])


def _run_heads(wcomb, eo, CW, Cb, NW, Nb, bt):
    grid = (_B // bt,)
    full = lambda *shape: pl.BlockSpec(shape, lambda b: (0,) * len(shape))
    return pl.pallas_call(
        _heads_body,
        grid=grid,
        in_specs=[
            pl.BlockSpec((bt, _S, _E), lambda b: (b, 0, 0)),
            pl.BlockSpec((_E, bt, _D), lambda b: (0, b, 0)),
            full(_NC, _D, _V),
            full(_NC, 1, _V),
            full(_NN, _D, 1),
            full(_NN, 1),
        ],
        out_specs=[
            pl.BlockSpec((bt, _NC, _V), lambda b: (b, 0, 0)),
            pl.BlockSpec((bt, _NN), lambda b: (b, 0)),
        ],
        out_shape=[
            jax.ShapeDtypeStruct((_B, _NC, _V), jnp.float32),
            jax.ShapeDtypeStruct((_B, _NN), jnp.float32),
        ],
    )(wcomb, eo, CW, Cb[:, None, :], NW, Nb)


def kernel(codec, mask_pos, pos_emb, type_emb, gW1, gb1, gW2, gb2, EW1, Eb1,
           EW2, Eb2, EW3, Eb3, NW, Nb, CW, Cb):
    maskf = mask_pos.astype(jnp.float32)
    logits = _run_gate(
        codec, maskf, pos_emb, type_emb, gW1, gb1, gW2, gb2, bt=256)
    gate_probs, topk_experts, wcomb = _run_routing(logits)
    eo = _run_experts(codec, EW1, Eb1, EW2, Eb2, EW3, Eb3)
    cat_recon, num_recon = _run_heads(wcomb, eo, CW, Cb, NW, Nb, bt=128)
    feature_indices = jnp.broadcast_to(
        jnp.arange(_S, dtype=jnp.int32)[None, :], (_B, _S))
    return (num_recon, cat_recon, gate_probs, topk_experts, mask_pos,
            feature_indices)
